# baseline (speedup 1.0000x reference)
import functools

import jax
import jax.numpy as jnp
import numpy as np
from jax import lax
from jax.experimental import pallas as pl
from jax.experimental.pallas import tpu as pltpu
from jax.experimental.pallas import tpu_sc as plsc

B = 16384
F = 512
OUT = 64
N = B * F
N_F = float(N)
DDOF = float(N / (N - 1.0))

NBINS = 65536
ROWS_PER_STEP = 1024
K1_STEPS = B // ROWS_PER_STEP

NWORKERS = 32
PER_W = N // NWORKERS
CROWS = 16
NCHUNK = (B // NWORKERS) // CROWS


def _k1_body(x_ref, mom_ref):
    i = pl.program_id(0)
    xb = x_ref[...]
    x2 = xb * xb

    def fold(v, op):
        return op(v.reshape(4, 128), axis=0)

    s1 = fold(jnp.sum(xb, axis=0), jnp.sum)
    s2 = fold(jnp.sum(x2, axis=0), jnp.sum)
    s3 = fold(jnp.sum(x2 * xb, axis=0), jnp.sum)
    s4 = fold(jnp.sum(x2 * x2, axis=0), jnp.sum)
    mn = fold(jnp.min(xb, axis=0), jnp.min)
    mx = fold(jnp.max(xb, axis=0), jnp.max)
    part = jnp.stack([s1, s2, s3, s4, mn, mx, mn, mx])

    @pl.when(i == 0)
    def _():
        mom_ref[...] = part

    @pl.when(i != 0)
    def _():
        old = mom_ref[...]
        mom_ref[...] = jnp.concatenate(
            [
                old[0:4] + part[0:4],
                jnp.minimum(old[4:5], part[4:5]),
                jnp.maximum(old[5:6], part[5:6]),
                old[6:8],
            ],
            axis=0,
        )


def _k1(x):
    return pl.pallas_call(
        _k1_body,
        grid=(K1_STEPS,),
        in_specs=[pl.BlockSpec((ROWS_PER_STEP, F), lambda i: (i, 0))],
        out_specs=pl.BlockSpec((8, 128), lambda i: (0, 0)),
        out_shape=jax.ShapeDtypeStruct((8, 128), jnp.float32),
    )(x)


def _k1b_body(mom_ref, par_ref):
    acc = mom_ref[...]
    lo = jnp.min(acc[4])
    hi = jnp.max(acc[5])
    scale = NBINS / jnp.maximum(hi - lo, 1e-30)
    par_ref[...] = jnp.concatenate(
        [jnp.full((1, 128), lo, jnp.float32),
         jnp.full((1, 128), scale, jnp.float32)],
        axis=0,
    )


def _k1b(mom):
    return pl.pallas_call(
        _k1b_body,
        out_shape=jax.ShapeDtypeStruct((2, 128), jnp.float32),
    )(mom)


def _sc_hist_body(x_hbm, par_hbm, out_hbm, hist, buf0, buf1, lobuf, scbuf, sem0, sem1):
    wid = lax.axis_index("s") * 2 + lax.axis_index("c")
    base_row = wid * (B // NWORKERS)

    pltpu.async_copy(x_hbm.at[pl.ds(base_row, CROWS), :], buf0, sem0)
    pltpu.async_copy(x_hbm.at[pl.ds(base_row + CROWS, CROWS), :], buf1, sem1)

    pltpu.sync_copy(par_hbm.at[pl.ds(0, 16)], lobuf)
    pltpu.sync_copy(par_hbm.at[pl.ds(128, 16)], scbuf)
    lo_v = lobuf[...]
    sc_v = scbuf[...]
    ones = jnp.ones((16,), jnp.float32)
    maxbin = jnp.float32(NBINS - 1)

    @plsc.parallel_loop(0, NBINS // 128, unroll=2)
    def _(i):
        for u in range(8):
            hist[pl.ds(i * 128 + u * 16, 16)] = jnp.zeros((16,), jnp.float32)

    def process(buf):
        for r in range(CROWS):
            @plsc.parallel_loop(0, F // 64, unroll=2)
            def _(c):
                for u in range(4):
                    v = buf[r, pl.ds(c * 64 + u * 16, 16)]
                    t = jnp.clip((v - lo_v) * sc_v, 0.0, maxbin)
                    plsc.addupdate_scatter(hist, [t.astype(jnp.int32)], ones)

    def pair(jj, _):
        r0 = base_row + (2 * jj) * CROWS
        pltpu.make_async_copy(x_hbm.at[pl.ds(r0, CROWS), :], buf0, sem0).wait()
        process(buf0)

        @pl.when(jj < NCHUNK // 2 - 1)
        def _():
            pltpu.async_copy(
                x_hbm.at[pl.ds(r0 + 2 * CROWS, CROWS), :], buf0, sem0)

        pltpu.make_async_copy(
            x_hbm.at[pl.ds(r0 + CROWS, CROWS), :], buf1, sem1).wait()
        process(buf1)

        @pl.when(jj < NCHUNK // 2 - 1)
        def _():
            pltpu.async_copy(
                x_hbm.at[pl.ds(r0 + 3 * CROWS, CROWS), :], buf1, sem1)

        return 0

    lax.fori_loop(0, NCHUNK // 2, pair, 0)

    pltpu.sync_copy(hist, out_hbm.at[wid])


@functools.cache
def _sc_hist():
    mesh = plsc.VectorSubcoreMesh(core_axis_name="c", subcore_axis_name="s")
    return pl.kernel(
        _sc_hist_body,
        out_type=jax.ShapeDtypeStruct((NWORKERS, NBINS), jnp.float32),
        mesh=mesh,
        scratch_types=[
            pltpu.VMEM((NBINS,), jnp.float32),
            pltpu.VMEM((CROWS, F), jnp.float32),
            pltpu.VMEM((CROWS, F), jnp.float32),
            pltpu.VMEM((16,), jnp.float32),
            pltpu.VMEM((16,), jnp.float32),
            pltpu.SemaphoreType.DMA,
            pltpu.SemaphoreType.DMA,
        ],
        compiler_params=pltpu.CompilerParams(needs_layout_passes=False),
    )


def _k3_body(mom_ref, hist_ref, sw_ref, W1_ref, b1_ref, W2_ref, b2_ref, out_ref):
    acc = mom_ref[...]
    s1 = jnp.sum(acc[0])
    s2 = jnp.sum(acc[1])
    s3 = jnp.sum(acc[2])
    s4 = jnp.sum(acc[3])
    lo = jnp.min(acc[4])
    hi = jnp.max(acc[5])
    width = jnp.maximum(hi - lo, 1e-30) / NBINS

    h = jnp.sum(hist_ref[...], axis=0).reshape(512, 128)

    cs = h
    k = 1
    while k < 128:
        cs = cs + jnp.concatenate(
            [jnp.zeros((512, k), jnp.float32), cs[:, :-k]], axis=1)
        k *= 2
    rt = cs[:, 127:128]
    rtc = rt
    k = 1
    while k < 512:
        rtc = rtc + jnp.concatenate(
            [jnp.zeros((k, 1), jnp.float32), rtc[:-k, :]], axis=0)
        k *= 2
    cdf = cs + (rtc - rt)

    gi = lax.broadcasted_iota(jnp.int32, (512, 128), 0)
    gj = lax.broadcasted_iota(jnp.int32, (512, 128), 1)
    gidx = (gi * 128 + gj).astype(jnp.float32)

    def orderstat(k):
        kf = jnp.float32(k)
        isge = cdf >= kf
        b = jnp.min(jnp.where(isge, gidx, jnp.float32(NBINS)))
        cdfp = jnp.max(jnp.where(isge, 0.0, cdf))
        cdfb = jnp.min(jnp.where(isge, cdf, jnp.float32(2 * N)))
        cnt = jnp.maximum(cdfb - cdfp, 1.0)
        frac = (kf - cdfp - 0.5) / cnt
        return lo + (b + frac) * width

    q25 = orderstat(N // 4 + 1)
    med_a = orderstat(N // 2)
    med_b = orderstat(N // 2 + 1)
    q75 = orderstat(3 * N // 4 + 1)

    mean = s1 / N_F
    m2r = s2 / N_F
    m3r = s3 / N_F
    m4r = s4 / N_F
    var_s = (m2r - mean * mean) * DDOF
    std = jnp.sqrt(jnp.maximum(var_s, 0.0)) + 1e-8
    var_o = var_s + 1e-8
    c3 = m3r - 3.0 * mean * m2r + 2.0 * mean * mean * mean
    c4 = m4r - 4.0 * mean * m3r + 6.0 * mean * mean * m2r - 3.0 * mean ** 4
    skew = c3 / (std * std * std + 1e-8)
    kurt = c4 / (std * std * std * std + 1e-8) - 3.0
    rng = hi - lo
    median = 0.5 * (med_a + med_b)
    iqr = q75 - q25

    stats = jnp.stack([mean, median, std, var_o, skew, kurt, rng, iqr])
    w = stats * sw_ref[...]
    h1 = jax.nn.relu(jnp.sum(W1_ref[...] * w[None, :], axis=1) + b1_ref[...])
    o = jnp.sum(W2_ref[...] * h1[None, :], axis=1) + b2_ref[...]
    out_ref[...] = jnp.broadcast_to(o[None, :], (B, OUT))


def _k3(mom, hist, sw, W1, b1, W2, b2):
    return pl.pallas_call(
        _k3_body,
        out_shape=jax.ShapeDtypeStruct((B, OUT), jnp.float32),
    )(mom, hist, sw, W1, b1, W2, b2)


def kernel(x, stat_weights, W1, b1, W2, b2):
    mom = _k1(x)
    par = _k1b(mom)
    hist = _sc_hist()(x, par.reshape(-1))
    return _k3(mom, hist, stat_weights, W1, b1, W2, b2)


# speedup vs baseline: 1.2870x; 1.2870x over previous
import functools

import jax
import jax.numpy as jnp
from jax import lax
from jax.experimental import pallas as pl
from jax.experimental.pallas import tpu as pltpu
from jax.experimental.pallas import tpu_sc as plsc

B = 16384
F = 512
OUT = 64
N = B * F
N_F = float(N)
DDOF = float(N / (N - 1.0))

NBINS = 65536
ROWS_PER_STEP = 1024
K1_STEPS = B // ROWS_PER_STEP

NWORKERS = 32
PER_W = N // NWORKERS
CROWS = 16
NCHUNK = (B // NWORKERS) // CROWS


def _k1_body(x_ref, mom_ref):
    i = pl.program_id(0)
    xb = x_ref[...]
    x2 = xb * xb

    def fold(v, op):
        return op(v.reshape(4, 128), axis=0)

    s1 = fold(jnp.sum(xb, axis=0), jnp.sum)
    s2 = fold(jnp.sum(x2, axis=0), jnp.sum)
    s3 = fold(jnp.sum(x2 * xb, axis=0), jnp.sum)
    s4 = fold(jnp.sum(x2 * x2, axis=0), jnp.sum)
    mn = fold(jnp.min(xb, axis=0), jnp.min)
    mx = fold(jnp.max(xb, axis=0), jnp.max)
    part = jnp.stack([s1, s2, s3, s4, mn, mx, mn, mx])

    @pl.when(i == 0)
    def _():
        mom_ref[...] = part

    @pl.when(i != 0)
    def _():
        old = mom_ref[...]
        mom_ref[...] = jnp.concatenate(
            [
                old[0:4] + part[0:4],
                jnp.minimum(old[4:5], part[4:5]),
                jnp.maximum(old[5:6], part[5:6]),
                old[6:8],
            ],
            axis=0,
        )


def _k1(x):
    return pl.pallas_call(
        _k1_body,
        grid=(K1_STEPS,),
        in_specs=[pl.BlockSpec((ROWS_PER_STEP, F), lambda i: (i, 0))],
        out_specs=pl.BlockSpec((8, 128), lambda i: (0, 0)),
        out_shape=jax.ShapeDtypeStruct((8, 128), jnp.float32),
    )(x)


def _k1b_body(mom_ref, par_ref):
    acc = mom_ref[...]
    lo = jnp.min(acc[4])
    hi = jnp.max(acc[5])
    scale = NBINS / jnp.maximum(hi - lo, 1e-30)
    par_ref[...] = jnp.concatenate(
        [jnp.full((1, 128), lo, jnp.float32),
         jnp.full((1, 128), scale, jnp.float32)],
        axis=0,
    )


def _k1b(mom):
    return pl.pallas_call(
        _k1b_body,
        out_shape=jax.ShapeDtypeStruct((2, 128), jnp.float32),
    )(mom)


def _sc_hist_body(x_hbm, par_hbm, out_hbm, hist, buf0, buf1, lobuf, scbuf, sem0, sem1):
    wid = lax.axis_index("s") * 2 + lax.axis_index("c")
    base_row = wid * (B // NWORKERS)

    pltpu.async_copy(x_hbm.at[pl.ds(base_row, CROWS), :], buf0, sem0)
    pltpu.async_copy(x_hbm.at[pl.ds(base_row + CROWS, CROWS), :], buf1, sem1)

    pltpu.sync_copy(par_hbm.at[pl.ds(0, 16)], lobuf)
    pltpu.sync_copy(par_hbm.at[pl.ds(128, 16)], scbuf)
    lo_v = lobuf[...]
    sc_v = scbuf[...]
    ones = jnp.ones((16,), jnp.float32)
    maxbin = jnp.float32(NBINS - 1)

    @plsc.parallel_loop(0, NBINS // 128, unroll=2)
    def _(i):
        for u in range(8):
            hist[pl.ds(i * 128 + u * 16, 16)] = jnp.zeros((16,), jnp.float32)

    def process(buf):
        for r in range(CROWS):
            @plsc.parallel_loop(0, F // 64, unroll=4)
            def _(c):
                for u in range(4):
                    v = buf[r, pl.ds(c * 64 + u * 16, 16)]
                    t = jnp.clip((v - lo_v) * sc_v, 0.0, maxbin)
                    plsc.addupdate_scatter(hist, [t.astype(jnp.int32)], ones)

    def pair(jj, _):
        r0 = base_row + (2 * jj) * CROWS
        pltpu.make_async_copy(x_hbm.at[pl.ds(r0, CROWS), :], buf0, sem0).wait()
        process(buf0)

        @pl.when(jj < NCHUNK // 2 - 1)
        def _():
            pltpu.async_copy(
                x_hbm.at[pl.ds(r0 + 2 * CROWS, CROWS), :], buf0, sem0)

        pltpu.make_async_copy(
            x_hbm.at[pl.ds(r0 + CROWS, CROWS), :], buf1, sem1).wait()
        process(buf1)

        @pl.when(jj < NCHUNK // 2 - 1)
        def _():
            pltpu.async_copy(
                x_hbm.at[pl.ds(r0 + 3 * CROWS, CROWS), :], buf1, sem1)

        return 0

    lax.fori_loop(0, NCHUNK // 2, pair, 0)

    pltpu.sync_copy(hist, out_hbm.at[wid])


@functools.cache
def _sc_hist():
    mesh = plsc.VectorSubcoreMesh(core_axis_name="c", subcore_axis_name="s")
    return pl.kernel(
        _sc_hist_body,
        out_type=jax.ShapeDtypeStruct((NWORKERS, NBINS), jnp.float32),
        mesh=mesh,
        scratch_types=[
            pltpu.VMEM((NBINS,), jnp.float32),
            pltpu.VMEM((CROWS, F), jnp.float32),
            pltpu.VMEM((CROWS, F), jnp.float32),
            pltpu.VMEM((16,), jnp.float32),
            pltpu.VMEM((16,), jnp.float32),
            pltpu.SemaphoreType.DMA,
            pltpu.SemaphoreType.DMA,
        ],
        compiler_params=pltpu.CompilerParams(needs_layout_passes=False),
    )


def _k3_body(mom_ref, hist_ref, sw_ref, W1_ref, b1_ref, W2_ref, b2_ref, out_ref):
    acc = mom_ref[...]
    s1 = jnp.sum(acc[0])
    s2 = jnp.sum(acc[1])
    s3 = jnp.sum(acc[2])
    s4 = jnp.sum(acc[3])
    lo = jnp.min(acc[4])
    hi = jnp.max(acc[5])
    width = jnp.maximum(hi - lo, 1e-30) / NBINS

    h = jnp.sum(hist_ref[...], axis=0).reshape(512, 128)

    cs = h
    k = 1
    while k < 128:
        cs = cs + jnp.concatenate(
            [jnp.zeros((512, k), jnp.float32), cs[:, :-k]], axis=1)
        k *= 2
    rt = cs[:, 127:128]
    rtc = rt
    k = 1
    while k < 512:
        rtc = rtc + jnp.concatenate(
            [jnp.zeros((k, 1), jnp.float32), rtc[:-k, :]], axis=0)
        k *= 2
    cdf = cs + (rtc - rt)

    gi = lax.broadcasted_iota(jnp.int32, (512, 128), 0)
    gj = lax.broadcasted_iota(jnp.int32, (512, 128), 1)
    gidx = (gi * 128 + gj).astype(jnp.float32)

    def orderstat(k):
        kf = jnp.float32(k)
        isge = cdf >= kf
        b = jnp.min(jnp.where(isge, gidx, jnp.float32(NBINS)))
        cdfp = jnp.max(jnp.where(isge, 0.0, cdf))
        cdfb = jnp.min(jnp.where(isge, cdf, jnp.float32(2 * N)))
        cnt = jnp.maximum(cdfb - cdfp, 1.0)
        frac = (kf - cdfp - 0.5) / cnt
        return lo + (b + frac) * width

    q25 = orderstat(N // 4 + 1)
    med_a = orderstat(N // 2)
    med_b = orderstat(N // 2 + 1)
    q75 = orderstat(3 * N // 4 + 1)

    mean = s1 / N_F
    m2r = s2 / N_F
    m3r = s3 / N_F
    m4r = s4 / N_F
    var_s = (m2r - mean * mean) * DDOF
    std = jnp.sqrt(jnp.maximum(var_s, 0.0)) + 1e-8
    var_o = var_s + 1e-8
    c3 = m3r - 3.0 * mean * m2r + 2.0 * mean * mean * mean
    c4 = m4r - 4.0 * mean * m3r + 6.0 * mean * mean * m2r - 3.0 * mean ** 4
    skew = c3 / (std * std * std + 1e-8)
    kurt = c4 / (std * std * std * std + 1e-8) - 3.0
    rng = hi - lo
    median = 0.5 * (med_a + med_b)
    iqr = q75 - q25

    stats = jnp.stack([mean, median, std, var_o, skew, kurt, rng, iqr])
    w = stats * sw_ref[...]
    h1 = jax.nn.relu(jnp.sum(W1_ref[...] * w[None, :], axis=1) + b1_ref[...])
    o = jnp.sum(W2_ref[...] * h1[None, :], axis=1) + b2_ref[...]
    out_ref[...] = jnp.broadcast_to(o[None, :], (B, OUT))


def _k3(mom, hist, sw, W1, b1, W2, b2):
    return pl.pallas_call(
        _k3_body,
        out_shape=jax.ShapeDtypeStruct((B, OUT), jnp.float32),
    )(mom, hist, sw, W1, b1, W2, b2)


def kernel(x, stat_weights, W1, b1, W2, b2):
    mom = _k1(x)
    par = _k1b(mom)
    hist = _sc_hist()(x, par.reshape(-1))
    return _k3(mom, hist, stat_weights, W1, b1, W2, b2)
